# baseline (speedup 1.0000x reference)
import functools

import jax
import jax.numpy as jnp
from jax import lax
from jax.experimental import pallas as pl
from jax.experimental.pallas import tpu as pltpu
from jax.experimental.pallas import tpu_sc as plsc

_B = 1024
_V = 100000
_NS = 16
_L = 16
_PER_W = _B // _NS
_CH = _PER_W // _L

_LN2 = 0.6931471805599453
_SQRT2 = 1.4142135623730951


def _neg_log(v):
    bits = lax.bitcast_convert_type(v, jnp.int32)
    e = lax.shift_right_arithmetic(bits, jnp.int32(23)) - 127
    m_bits = lax.bitwise_or(
        lax.bitwise_and(bits, jnp.int32(0x007FFFFF)), jnp.int32(0x3F800000)
    )
    m = lax.bitcast_convert_type(m_bits, jnp.float32)
    big = m >= _SQRT2
    m = jnp.where(big, m * 0.5, m)
    e = jnp.where(big, e + 1, e)
    ef = e.astype(jnp.float32)
    s = (m - 1.0) / (m + 1.0)
    z = s * s
    p = s * (2.0 + z * (2.0 / 3.0 + z * (2.0 / 5.0 + z * (2.0 / 7.0 + z * (2.0 / 9.0)))))
    return -(ef * _LN2 + p)


_mesh = plsc.VectorSubcoreMesh(
    core_axis_name="c", subcore_axis_name="s", num_cores=1
)


@functools.partial(
    pl.kernel,
    mesh=_mesh,
    out_type=(
        jax.ShapeDtypeStruct((_NS, _L), jnp.float32),
        jax.ShapeDtypeStruct((_L,), jnp.float32),
    ),
    scratch_types=[
        pltpu.VMEM((_PER_W,), jnp.int32),
        pltpu.VMEM((_PER_W,), jnp.int32),
        pltpu.VMEM((_PER_W,), jnp.float32),
        pltpu.VMEM((_L,), jnp.float32),
        pltpu.VMEM((_NS, _L), jnp.float32),
        pltpu.SemaphoreType.DMA,
    ],
)
def _nll_sc(x_hbm, y_hbm, parts_hbm, out_hbm, y_v, idx_v, val_v, part_v, all_v, sem):
    w = lax.axis_index("s")
    base = w * _PER_W
    pltpu.sync_copy(y_hbm.at[pl.ds(base, _PER_W)], y_v)
    lanes = lax.iota(jnp.int32, _L)
    for t in range(_CH):
        rows = base + t * _L + lanes
        idx_v[pl.ds(t * _L, _L)] = rows * _V + y_v[pl.ds(t * _L, _L)]
    pltpu.async_copy(x_hbm.at[idx_v], val_v, sem).wait()
    acc = _neg_log(val_v[pl.ds(0, _L)])
    for t in range(1, _CH):
        acc = acc + _neg_log(val_v[pl.ds(t * _L, _L)])
    part_v[...] = acc
    pltpu.sync_copy(part_v, parts_hbm.at[w])
    plsc.subcore_barrier()

    @pl.when(w == 0)
    def _():
        pltpu.sync_copy(parts_hbm, all_v)
        tot = all_v[0]
        for i in range(1, _NS):
            tot = tot + all_v[i]
        s = tot[0]
        for i in range(1, _L):
            s = s + tot[i]
        part_v[...] = jnp.full((_L,), s * (1.0 / _B), jnp.float32)
        pltpu.sync_copy(part_v, out_hbm)


def kernel(x, y):
    _, out = _nll_sc(x.reshape(-1), y)
    return out[0]


# speedup vs baseline: 2.3447x; 2.3447x over previous
import functools

import jax
import jax.numpy as jnp
from jax import lax
from jax.experimental import pallas as pl
from jax.experimental.pallas import tpu as pltpu
from jax.experimental.pallas import tpu_sc as plsc

_B = 1024
_V = 100000
_NS = 16
_L = 16
_PER_W = _B // _NS
_CH = _PER_W // _L

_LN2 = 0.6931471805599453
_SQRT2 = 1.4142135623730951


def _neg_log(v):
    bits = lax.bitcast_convert_type(v, jnp.int32)
    e = lax.shift_right_arithmetic(bits, jnp.int32(23)) - 127
    m_bits = lax.bitwise_or(
        lax.bitwise_and(bits, jnp.int32(0x007FFFFF)), jnp.int32(0x3F800000)
    )
    m = lax.bitcast_convert_type(m_bits, jnp.float32)
    big = m >= _SQRT2
    m = jnp.where(big, m * 0.5, m)
    e = jnp.where(big, e + 1, e)
    ef = e.astype(jnp.float32)
    s = (m - 1.0) / (m + 1.0)
    z = s * s
    p = s * (2.0 + z * (2.0 / 3.0 + z * (2.0 / 5.0 + z * (2.0 / 7.0 + z * (2.0 / 9.0)))))
    return -(ef * _LN2 + p)


_mesh = plsc.VectorSubcoreMesh(
    core_axis_name="c", subcore_axis_name="s", num_cores=1
)


@functools.partial(
    pl.kernel,
    mesh=_mesh,
    out_type=(
        jax.ShapeDtypeStruct((_NS, 8, 128), jnp.float32),
        jax.ShapeDtypeStruct((8, 128), jnp.float32),
    ),
    scratch_types=[
        pltpu.VMEM((_B,), jnp.int32),
        pltpu.VMEM((_PER_W, 8, 128), jnp.float32),
        pltpu.VMEM((8, 128), jnp.float32),
        pltpu.VMEM((_NS, 8, 128), jnp.float32),
        pltpu.SemaphoreType.DMA,
    ],
    compiler_params=pltpu.CompilerParams(
        use_tc_tiling_on_sc=True, needs_layout_passes=False
    ),
)
def _nll_sc(x_hbm, y_hbm, parts_hbm, out_hbm, y_v, tiles_v, buf_v, all_v, sem):
    w = lax.axis_index("s")
    base = w * _PER_W
    pltpu.sync_copy(y_hbm, y_v)
    cps = []
    for t in range(_CH):
        yv = y_v[pl.ds(base + t * _L, _L)]
        for u in range(_L):
            j = t * _L + u
            ys = yv[u]
            col0 = pl.multiple_of((ys // 128) * 128, 128)
            row0 = pl.multiple_of(base + (j // 8) * 8, 8)
            cps.append(pltpu.async_copy(
                x_hbm.at[pl.ds(row0, 8), pl.ds(col0, 128)], tiles_v.at[j], sem))
    for cp in cps:
        cp.wait()
    lanes = lax.iota(jnp.int32, _L)
    acc = None
    for t in range(_CH):
        j_vec = lanes + t * _L
        r_vec = lax.rem(j_vec, 8)
        c_vec = lax.bitwise_and(y_v[pl.ds(base + t * _L, _L)], jnp.int32(127))
        val = plsc.load_gather(tiles_v, [j_vec, r_vec, c_vec])
        nl = _neg_log(val)
        acc = nl if acc is None else acc + nl
    buf_v[0, pl.ds(0, _L)] = acc
    pltpu.sync_copy(buf_v, parts_hbm.at[w])
    plsc.subcore_barrier()

    @pl.when(w == 0)
    def _():
        pltpu.sync_copy(parts_hbm, all_v)
        tot = all_v[0, 0, pl.ds(0, _L)]
        for i in range(1, _NS):
            tot = tot + all_v[i, 0, pl.ds(0, _L)]
        s = tot[0]
        for i in range(1, _L):
            s = s + tot[i]
        buf_v[0, pl.ds(0, _L)] = jnp.full((_L,), s * (1.0 / _B), jnp.float32)
        pltpu.sync_copy(buf_v, out_hbm)


def kernel(x, y):
    _, out = _nll_sc(x, y)
    return out[0, 0]
